# baseline (speedup 1.0000x reference)
import functools

import jax
import jax.numpy as jnp
from jax import lax
from jax.experimental import pallas as pl
from jax.experimental.pallas import tpu as pltpu
from jax.experimental.pallas import tpu_sc as plsc

_TOP_K = 8
_NW = 32
_L = 16


def _gelu_exact(v):
    return v * 0.5 * (1.0 + jax.lax.erf(v * 0.7071067811865476))


def _logits_kernel(x_ref, w1_ref, b1_ref, w2_ref, b2_ref, out_ref, acc_ref,
                   *, j_steps):
    j = pl.program_id(1)

    h = jnp.dot(x_ref[...], w1_ref[...], preferred_element_type=jnp.float32)
    h = _gelu_exact(h + b1_ref[...])
    part = jnp.dot(h, w2_ref[...], preferred_element_type=jnp.float32)

    @pl.when(j == 0)
    def _():
        acc_ref[...] = part

    @pl.when(j > 0)
    def _():
        acc_ref[...] = acc_ref[...] + part

    @pl.when(j == j_steps - 1)
    def _():
        out_ref[...] = jnp.transpose(acc_ref[...] + b2_ref[...])


def _route_kernel(lg_hbm, rout_hbm, idx_hbm, stats_hbm,
                  lg_v, rout_v, idxo_v, sbuf_v, stats_v,
                  *, rows_w, num_experts):
    wid = lax.axis_index("c") * 16 + lax.axis_index("s")
    base = wid * rows_w
    groups = rows_w // _L
    zeros = jnp.zeros((_L,), jnp.float32)

    pltpu.sync_copy(lg_hbm.at[:, pl.ds(base, rows_w)], lg_v)

    def _zero_stats(i, _):
        stats_v[i, pl.ds(0, _L)] = zeros
        return 0
    lax.fori_loop(0, 2 * num_experts, _zero_stats, 0)

    def _group(g, _):
        col = g * _L

        def _expert(e, slots):
            sv, si = slots
            cand_v = lg_v[e, pl.ds(col, _L)]
            cand_i = jnp.zeros((_L,), jnp.int32) + e
            new_v, new_i = [], []
            for s in range(_TOP_K):
                beats = cand_v > sv[s]
                new_v.append(jnp.where(beats, cand_v, sv[s]))
                new_i.append(jnp.where(beats, cand_i, si[s]))
                cand_v = jnp.where(beats, sv[s], cand_v)
                cand_i = jnp.where(beats, si[s], cand_i)
            return tuple(new_v), tuple(new_i)

        init = (tuple(jnp.full((_L,), -jnp.inf, jnp.float32)
                      for _ in range(_TOP_K)),
                tuple(jnp.zeros((_L,), jnp.int32) for _ in range(_TOP_K)))
        top_v, top_i = lax.fori_loop(0, num_experts, _expert, init)

        ex = [jnp.exp(v - top_v[0]) for v in top_v]
        tinv = 1.0 / functools.reduce(lambda a, b: a + b, ex)
        for k in range(_TOP_K):
            idxo_v[k, pl.ds(col, _L)] = top_i[k]

        def _p2(e, acc):
            s_e = jnp.exp(lg_v[e, pl.ds(col, _L)] - top_v[0])
            sbuf_v[e, pl.ds(0, _L)] = s_e
            return acc + s_e
        fsum = lax.fori_loop(0, num_experts, _p2, zeros)
        finv = 1.0 / fsum

        def _p3(e, _):
            member = top_i[0] == e
            for k in range(1, _TOP_K):
                member = jnp.logical_or(member, top_i[k] == e)
            s_e = sbuf_v[e, pl.ds(0, _L)]
            memf = jnp.where(member, 1.0, 0.0)
            rout_v[e, pl.ds(col, _L)] = s_e * tinv * memf
            stats_v[e, pl.ds(0, _L)] = (stats_v[e, pl.ds(0, _L)]
                                        + s_e * finv)
            fr = num_experts + e
            stats_v[fr, pl.ds(0, _L)] = stats_v[fr, pl.ds(0, _L)] + memf
            return 0
        lax.fori_loop(0, num_experts, _p3, 0)
        return 0

    lax.fori_loop(0, groups, _group, 0)

    pltpu.sync_copy(rout_v, rout_hbm.at[:, pl.ds(base, rows_w)])
    pltpu.sync_copy(idxo_v, idx_hbm.at[:, pl.ds(base, rows_w)])
    pltpu.sync_copy(stats_v, stats_hbm.at[wid])


def _loss_kernel(stats_ref, loss_ref, *, n_rows, num_experts):
    s = jnp.sum(stats_ref[...], axis=(0, 2))
    psum = s[:num_experts]
    msum = s[num_experts:]
    inv_n2 = 1.0 / (float(n_rows) * float(n_rows))
    loss_ref[...] = (float(num_experts) * inv_n2
                     * jnp.sum(psum * msum, keepdims=True).reshape(1, 1))


def kernel(x, W1, b1, W2, b2, noise_scale):
    del noise_scale
    n, d = x.shape
    e = W2.shape[1]
    rows_blk = min(1024, n)
    j_blk = min(512, d)
    r_steps = n // rows_blk
    j_steps = d // j_blk
    rows_w = n // _NW

    logits_t = pl.pallas_call(
        functools.partial(_logits_kernel, j_steps=j_steps),
        grid=(r_steps, j_steps),
        in_specs=[
            pl.BlockSpec((rows_blk, d), lambda r, j: (r, 0)),
            pl.BlockSpec((d, j_blk), lambda r, j: (0, j)),
            pl.BlockSpec((1, j_blk), lambda r, j: (0, j)),
            pl.BlockSpec((j_blk, e), lambda r, j: (j, 0)),
            pl.BlockSpec((1, e), lambda r, j: (0, 0)),
        ],
        out_specs=pl.BlockSpec((e, rows_blk), lambda r, j: (0, r)),
        out_shape=jax.ShapeDtypeStruct((e, n), jnp.float32),
        scratch_shapes=[pltpu.VMEM((rows_blk, e), jnp.float32)],
        compiler_params=pltpu.CompilerParams(
            dimension_semantics=("parallel", "arbitrary"),
        ),
    )(x, W1, b1.reshape(1, d), W2, b2.reshape(1, e))

    mesh = plsc.VectorSubcoreMesh(core_axis_name="c", subcore_axis_name="s")
    route = pl.kernel(
        functools.partial(_route_kernel, rows_w=rows_w, num_experts=e),
        out_type=[
            jax.ShapeDtypeStruct((e, n), jnp.float32),
            jax.ShapeDtypeStruct((_TOP_K, n), jnp.int32),
            jax.ShapeDtypeStruct((_NW, 2 * e, _L), jnp.float32),
        ],
        mesh=mesh,
        scratch_types=[
            pltpu.VMEM((e, rows_w), jnp.float32),
            pltpu.VMEM((e, rows_w), jnp.float32),
            pltpu.VMEM((_TOP_K, rows_w), jnp.int32),
            pltpu.VMEM((e, _L), jnp.float32),
            pltpu.VMEM((2 * e, _L), jnp.float32),
        ],
    )
    rout_t, idx_t, stats = route(logits_t)

    loss = pl.pallas_call(
        functools.partial(_loss_kernel, n_rows=n, num_experts=e),
        out_shape=jax.ShapeDtypeStruct((1, 1), jnp.float32),
    )(stats)

    return (jnp.transpose(rout_t), jnp.transpose(idx_t), loss.reshape(()))


# speedup vs baseline: 1.1092x; 1.1092x over previous
import functools

import jax
import jax.numpy as jnp
from jax.experimental import pallas as pl
from jax.experimental.pallas import tpu as pltpu

_TOP_K = 8


def _gelu_exact(v):
    return v * 0.5 * (1.0 + jax.lax.erf(v * 0.7071067811865476))


def _router_kernel(x_ref, w1_ref, b1_ref, w2_ref, b2_ref,
                   rout_ref, idx_ref, stats_ref,
                   acc_ref,
                   *, j_steps, num_experts):
    j = pl.program_id(1)

    h = jnp.dot(x_ref[...], w1_ref[...], preferred_element_type=jnp.float32)
    h = _gelu_exact(h + b1_ref[...])
    part = jnp.dot(h, w2_ref[...], preferred_element_type=jnp.float32)

    @pl.when(j == 0)
    def _():
        acc_ref[...] = part

    @pl.when(j > 0)
    def _():
        acc_ref[...] = acc_ref[...] + part

    @pl.when(j == j_steps - 1)
    def _():
        logits = acc_ref[...] + b2_ref[...]
        rows = logits.shape[0]
        rowmax = jnp.max(logits, axis=1, keepdims=True)
        e = jnp.exp(logits - rowmax)
        probs = e / jnp.sum(e, axis=1, keepdims=True)

        iota = jax.lax.broadcasted_iota(jnp.int32, (rows, num_experts), 1)
        work = logits
        topmask = jnp.zeros((rows, num_experts), dtype=jnp.bool_)
        idx_cols = []
        for _k in range(_TOP_K):
            m = jnp.max(work, axis=1, keepdims=True)
            idx = jnp.min(jnp.where(work == m, iota, num_experts),
                          axis=1, keepdims=True)
            sel = iota == idx
            topmask = jnp.logical_or(topmask, sel)
            idx_cols.append(idx)
            work = jnp.where(sel, -jnp.inf, work)
        idx_ref[...] = jnp.concatenate(idx_cols, axis=1)

        te = jnp.where(topmask, e, 0.0)
        rout_ref[...] = te / jnp.sum(te, axis=1, keepdims=True)

        prob_part = jnp.sum(probs, axis=0, keepdims=True)
        mask_part = jnp.sum(topmask.astype(jnp.float32), axis=0, keepdims=True)
        stats_ref[...] = jnp.stack([prob_part, mask_part], axis=1)


def _loss_kernel(stats_ref, loss_ref, *, n_rows, num_experts):
    psum = jnp.sum(stats_ref[:, 0, :], axis=0)
    msum = jnp.sum(stats_ref[:, 1, :], axis=0)
    inv_n2 = 1.0 / (float(n_rows) * float(n_rows))
    loss_ref[...] = (float(num_experts) * inv_n2
                     * jnp.sum(psum * msum, keepdims=True).reshape(1, 1))


def kernel(x, W1, b1, W2, b2, noise_scale):
    del noise_scale
    n, d = x.shape
    e = W2.shape[1]
    rows_blk = min(1024, n)
    j_blk = min(512, d)
    r_steps = n // rows_blk
    j_steps = d // j_blk

    body = functools.partial(
        _router_kernel, j_steps=j_steps, num_experts=e)

    rout, idx, stats = pl.pallas_call(
        body,
        grid=(r_steps, j_steps),
        in_specs=[
            pl.BlockSpec((rows_blk, d), lambda r, j: (r, 0)),
            pl.BlockSpec((d, j_blk), lambda r, j: (0, j)),
            pl.BlockSpec((1, j_blk), lambda r, j: (0, j)),
            pl.BlockSpec((j_blk, e), lambda r, j: (j, 0)),
            pl.BlockSpec((1, e), lambda r, j: (0, 0)),
        ],
        out_specs=[
            pl.BlockSpec((rows_blk, e), lambda r, j: (r, 0)),
            pl.BlockSpec((rows_blk, _TOP_K), lambda r, j: (r, 0)),
            pl.BlockSpec((1, 2, e), lambda r, j: (r, 0, 0)),
        ],
        out_shape=[
            jax.ShapeDtypeStruct((n, e), jnp.float32),
            jax.ShapeDtypeStruct((n, _TOP_K), jnp.int32),
            jax.ShapeDtypeStruct((r_steps, 2, e), jnp.float32),
        ],
        scratch_shapes=[
            pltpu.VMEM((rows_blk, e), jnp.float32),
        ],
        compiler_params=pltpu.CompilerParams(
            dimension_semantics=("parallel", "arbitrary"),
        ),
    )(x, W1, b1.reshape(1, d), W2, b2.reshape(1, e))

    loss = pl.pallas_call(
        functools.partial(_loss_kernel, n_rows=n, num_experts=e),
        out_shape=jax.ShapeDtypeStruct((1, 1), jnp.float32),
    )(stats)

    return rout, idx, loss.reshape(())
